# baseline (speedup 1.0000x reference)
import dataclasses
import functools

import jax
import jax.numpy as jnp
from jax import lax
from jax.experimental import pallas as pl
from jax.experimental.pallas import tpu as pltpu
from jax.experimental.pallas import tpu_sc as plsc

L = 16
CH = 48
NBUF = 4


def _sc_body(T_half, D, x_hbm, m_hbm, psum_hbm, pcnt_hbm,
             m_v, idx_v, rows, ics, acc_v, cnt_v, sems):
    nseg = D // L
    wid = lax.axis_index("s") * 2 + lax.axis_index("c")
    b = wid // 2
    half = wid % 2
    base_t = half * T_half
    grow = b * (2 * T_half) + base_t

    pltpu.sync_copy(m_hbm.at[b, pl.ds(base_t, T_half)], m_v)

    zi = jnp.zeros((L,), jnp.int32)

    @pl.loop(0, idx_v.shape[0] // L)
    def _(i):
        idx_v[pl.ds(i * L, L)] = zi

    iota = lax.iota(jnp.int32, L)

    def compact(i, off):
        mv = m_v[pl.ds(i * L, L)]
        msk = mv != 0
        gidx = grow + i * L + iota
        plsc.store_compressed(idx_v.at[pl.ds(off, L)], gidx, mask=msk)
        return off + jnp.sum(mv)

    n = lax.fori_loop(0, T_half // L, compact, jnp.int32(0))

    n_full = n // CH
    rem = n - n_full * CH

    def start(c, k):
        for j in range(CH // L):
            ics[k][pl.ds(j * L, L)] = idx_v[pl.ds(c * CH + j * L, L)]
        pltpu.async_copy(x_hbm.at[ics[k]], rows[k], sems[k])

    def wait(k):
        pltpu.make_async_copy(x_hbm.at[ics[k]], rows[k], sems[k]).wait()

    def accum_chunk(buf, nrows, accs):
        def row_add(j, a):
            return tuple(a[s] + buf[j, pl.ds(s * L, L)] for s in range(nseg))
        return lax.fori_loop(0, nrows, row_add, accs)

    zf = jnp.zeros((L,), jnp.float32)
    accs0 = (zf,) * nseg

    for k in range(NBUF):
        @pl.when(k < n_full)
        def _(k=k):
            start(k, k)

    def group(p, accs):
        for k in range(NBUF):
            c = p * NBUF + k

            def consume(a, k=k, c=c):
                wait(k)
                a = accum_chunk(rows[k], CH, a)

                @pl.when(c + NBUF < n_full)
                def _():
                    start(c + NBUF, k)

                return a

            accs = lax.cond(c < n_full, consume, lambda a: a, accs)
        return accs

    n_groups = (n_full + NBUF - 1) // NBUF
    accs = lax.fori_loop(0, n_groups, group, accs0)

    def tail(a):
        for j in range(CH // L):
            ics[0][pl.ds(j * L, L)] = idx_v[pl.ds(n_full * CH + j * L, L)]
        pltpu.sync_copy(x_hbm.at[ics[0]], rows[0])
        return accum_chunk(rows[0], rem, a)

    accs = lax.cond(rem > 0, tail, lambda a: a, accs)

    for s in range(nseg):
        acc_v[pl.ds(s * L, L)] = accs[s]

    pltpu.sync_copy(acc_v, psum_hbm.at[wid])
    cnt_v[pl.ds(0, L)] = jnp.full((L,), n, jnp.int32)
    pltpu.sync_copy(cnt_v, pcnt_hbm.at[wid])


def kernel(inputs, mask):
    B, T, D = inputs.shape
    T_half = T // 2
    NW = 32
    x2d = inputs.reshape(B * T, D)
    m32 = mask.astype(jnp.int32)

    mesh = plsc.VectorSubcoreMesh(core_axis_name="c", subcore_axis_name="s")
    cp = dataclasses.replace(pltpu.CompilerParams(), needs_layout_passes=False)
    sc = pl.kernel(
        functools.partial(_sc_body, T_half, D),
        out_type=(
            jax.ShapeDtypeStruct((NW, D), jnp.float32),
            jax.ShapeDtypeStruct((NW, L), jnp.int32),
        ),
        mesh=mesh,
        scratch_types=[
            pltpu.VMEM((T_half,), jnp.int32),
            pltpu.VMEM((T_half + 2 * CH,), jnp.int32),
            [pltpu.VMEM((CH, D), jnp.float32) for _ in range(NBUF)],
            [pltpu.VMEM((CH,), jnp.int32) for _ in range(NBUF)],
            pltpu.VMEM((D,), jnp.float32),
            pltpu.VMEM((L,), jnp.int32),
            [pltpu.SemaphoreType.DMA for _ in range(NBUF)],
        ],
        compiler_params=cp,
    )
    psum, pcnt = sc(x2d, m32)
    sums = psum.reshape(B, 2, D).sum(axis=1)
    counts = pcnt[:, 0].reshape(B, 2).sum(axis=1)
    return sums / counts[:, None].astype(inputs.dtype)


# speedup vs baseline: 1.1706x; 1.1706x over previous
import dataclasses
import functools

import jax
import jax.numpy as jnp
from jax import lax
from jax.experimental import pallas as pl
from jax.experimental.pallas import tpu as pltpu
from jax.experimental.pallas import tpu_sc as plsc

L = 16
CH = 32
NBUF = 6


def _sc_body(T_half, D, x_hbm, m_hbm, psum_hbm, pcnt_hbm,
             m_v, idx_v, rows, ics, acc_v, cnt_v, sems):
    nseg = D // L
    wid = lax.axis_index("s") * 2 + lax.axis_index("c")
    b = wid // 2
    half = wid % 2
    base_t = half * T_half
    grow = b * (2 * T_half) + base_t

    pltpu.sync_copy(m_hbm.at[b, pl.ds(base_t, T_half)], m_v)

    zi = jnp.zeros((L,), jnp.int32)

    @pl.loop(0, idx_v.shape[0] // L)
    def _(i):
        idx_v[pl.ds(i * L, L)] = zi

    iota = lax.iota(jnp.int32, L)

    def compact(i, off):
        mv = m_v[pl.ds(i * L, L)]
        msk = mv != 0
        gidx = grow + i * L + iota
        plsc.store_compressed(idx_v.at[pl.ds(off, L)], gidx, mask=msk)
        return off + jnp.sum(mv)

    n = lax.fori_loop(0, T_half // L, compact, jnp.int32(0))

    n_full = n // CH
    rem = n - n_full * CH

    def start(c, k):
        for j in range(CH // L):
            ics[k][pl.ds(j * L, L)] = idx_v[pl.ds(c * CH + j * L, L)]
        pltpu.async_copy(x_hbm.at[ics[k]], rows[k], sems[k])

    def wait(k):
        pltpu.make_async_copy(x_hbm.at[ics[k]], rows[k], sems[k]).wait()

    def accum_chunk(buf, nrows, accs):
        def row_add(j, a):
            return tuple(a[s] + buf[j, pl.ds(s * L, L)] for s in range(nseg))
        return lax.fori_loop(0, nrows, row_add, accs)

    zf = jnp.zeros((L,), jnp.float32)
    accs0 = (zf,) * nseg

    for k in range(NBUF):
        @pl.when(k < n_full)
        def _(k=k):
            start(k, k)

    def group(p, accs):
        for k in range(NBUF):
            c = p * NBUF + k

            def consume(a, k=k, c=c):
                wait(k)
                a = accum_chunk(rows[k], CH, a)

                @pl.when(c + NBUF < n_full)
                def _():
                    start(c + NBUF, k)

                return a

            accs = lax.cond(c < n_full, consume, lambda a: a, accs)
        return accs

    n_groups = (n_full + NBUF - 1) // NBUF
    accs = lax.fori_loop(0, n_groups, group, accs0)

    def tail(a):
        for j in range(CH // L):
            ics[0][pl.ds(j * L, L)] = idx_v[pl.ds(n_full * CH + j * L, L)]
        pltpu.sync_copy(x_hbm.at[ics[0]], rows[0])
        return accum_chunk(rows[0], rem, a)

    accs = lax.cond(rem > 0, tail, lambda a: a, accs)

    for s in range(nseg):
        acc_v[pl.ds(s * L, L)] = accs[s]

    pltpu.sync_copy(acc_v, psum_hbm.at[wid])
    cnt_v[pl.ds(0, L)] = jnp.full((L,), n, jnp.int32)
    pltpu.sync_copy(cnt_v, pcnt_hbm.at[wid])


def kernel(inputs, mask):
    B, T, D = inputs.shape
    T_half = T // 2
    NW = 32
    x2d = inputs.reshape(B * T, D)
    m32 = mask.astype(jnp.int32)

    mesh = plsc.VectorSubcoreMesh(core_axis_name="c", subcore_axis_name="s")
    cp = dataclasses.replace(pltpu.CompilerParams(), needs_layout_passes=False)
    sc = pl.kernel(
        functools.partial(_sc_body, T_half, D),
        out_type=(
            jax.ShapeDtypeStruct((NW, D), jnp.float32),
            jax.ShapeDtypeStruct((NW, L), jnp.int32),
        ),
        mesh=mesh,
        scratch_types=[
            pltpu.VMEM((T_half,), jnp.int32),
            pltpu.VMEM((T_half + 2 * CH,), jnp.int32),
            [pltpu.VMEM((CH, D), jnp.float32) for _ in range(NBUF)],
            [pltpu.VMEM((CH,), jnp.int32) for _ in range(NBUF)],
            pltpu.VMEM((D,), jnp.float32),
            pltpu.VMEM((L,), jnp.int32),
            [pltpu.SemaphoreType.DMA for _ in range(NBUF)],
        ],
        compiler_params=cp,
    )
    psum, pcnt = sc(x2d, m32)
    sums = psum.reshape(B, 2, D).sum(axis=1)
    counts = pcnt[:, 0].reshape(B, 2).sum(axis=1)
    return sums / counts[:, None].astype(inputs.dtype)


# speedup vs baseline: 1.3499x; 1.1532x over previous
import dataclasses
import functools

import jax
import jax.numpy as jnp
from jax import lax
from jax.experimental import pallas as pl
from jax.experimental.pallas import tpu as pltpu
from jax.experimental.pallas import tpu_sc as plsc

L = 16
CH = 16
NBUF = 12


def _sc_body(T_half, D, x_hbm, m_hbm, psum_hbm, pcnt_hbm,
             m_v, idx_v, rows, ics, acc_v, cnt_v, sems):
    nseg = D // L
    wid = lax.axis_index("s") * 2 + lax.axis_index("c")
    b = wid // 2
    half = wid % 2
    base_t = half * T_half
    grow = b * (2 * T_half) + base_t

    pltpu.sync_copy(m_hbm.at[b, pl.ds(base_t, T_half)], m_v)

    zi = jnp.zeros((L,), jnp.int32)

    @pl.loop(0, idx_v.shape[0] // L)
    def _(i):
        idx_v[pl.ds(i * L, L)] = zi

    iota = lax.iota(jnp.int32, L)

    def compact(i, off):
        mv = m_v[pl.ds(i * L, L)]
        msk = mv != 0
        gidx = grow + i * L + iota
        plsc.store_compressed(idx_v.at[pl.ds(off, L)], gidx, mask=msk)
        return off + jnp.sum(mv)

    n = lax.fori_loop(0, T_half // L, compact, jnp.int32(0))

    n_full = n // CH
    rem = n - n_full * CH

    def start(c, k):
        for j in range(CH // L):
            ics[k][pl.ds(j * L, L)] = idx_v[pl.ds(c * CH + j * L, L)]
        pltpu.async_copy(x_hbm.at[ics[k]], rows[k], sems[k])

    def wait(k):
        pltpu.make_async_copy(x_hbm.at[ics[k]], rows[k], sems[k]).wait()

    def accum_chunk(buf, nrows, accs):
        def row_add(j, a):
            return tuple(a[s] + buf[j, pl.ds(s * L, L)] for s in range(nseg))
        return lax.fori_loop(0, nrows, row_add, accs)

    zf = jnp.zeros((L,), jnp.float32)
    accs0 = (zf,) * nseg

    for k in range(NBUF):
        @pl.when(k < n_full)
        def _(k=k):
            start(k, k)

    def group(p, accs):
        for k in range(NBUF):
            c = p * NBUF + k

            def consume(a, k=k, c=c):
                wait(k)
                a = accum_chunk(rows[k], CH, a)

                @pl.when(c + NBUF < n_full)
                def _():
                    start(c + NBUF, k)

                return a

            accs = lax.cond(c < n_full, consume, lambda a: a, accs)
        return accs

    n_groups = (n_full + NBUF - 1) // NBUF
    accs = lax.fori_loop(0, n_groups, group, accs0)

    def tail(a):
        for j in range(CH // L):
            ics[0][pl.ds(j * L, L)] = idx_v[pl.ds(n_full * CH + j * L, L)]
        pltpu.sync_copy(x_hbm.at[ics[0]], rows[0])
        return accum_chunk(rows[0], rem, a)

    accs = lax.cond(rem > 0, tail, lambda a: a, accs)

    for s in range(nseg):
        acc_v[pl.ds(s * L, L)] = accs[s]

    pltpu.sync_copy(acc_v, psum_hbm.at[wid])
    cnt_v[pl.ds(0, L)] = jnp.full((L,), n, jnp.int32)
    pltpu.sync_copy(cnt_v, pcnt_hbm.at[wid])


def kernel(inputs, mask):
    B, T, D = inputs.shape
    T_half = T // 2
    NW = 32
    x2d = inputs.reshape(B * T, D)
    m32 = mask.astype(jnp.int32)

    mesh = plsc.VectorSubcoreMesh(core_axis_name="c", subcore_axis_name="s")
    cp = dataclasses.replace(pltpu.CompilerParams(), needs_layout_passes=False)
    sc = pl.kernel(
        functools.partial(_sc_body, T_half, D),
        out_type=(
            jax.ShapeDtypeStruct((NW, D), jnp.float32),
            jax.ShapeDtypeStruct((NW, L), jnp.int32),
        ),
        mesh=mesh,
        scratch_types=[
            pltpu.VMEM((T_half,), jnp.int32),
            pltpu.VMEM((T_half + 2 * CH,), jnp.int32),
            [pltpu.VMEM((CH, D), jnp.float32) for _ in range(NBUF)],
            [pltpu.VMEM((CH,), jnp.int32) for _ in range(NBUF)],
            pltpu.VMEM((D,), jnp.float32),
            pltpu.VMEM((L,), jnp.int32),
            [pltpu.SemaphoreType.DMA for _ in range(NBUF)],
        ],
        compiler_params=cp,
    )
    psum, pcnt = sc(x2d, m32)
    sums = psum.reshape(B, 2, D).sum(axis=1)
    counts = pcnt[:, 0].reshape(B, 2).sum(axis=1)
    return sums / counts[:, None].astype(inputs.dtype)
